# baseline (speedup 1.0000x reference)
import functools

import jax
import jax.numpy as jnp
from jax import lax
from jax.experimental import pallas as pl
from jax.experimental.pallas import tpu as pltpu
from jax.experimental.pallas import tpu_sc as plsc

BATCH = 16384
DIM = 32
NUM_CORES = 2
NUM_SUBCORES = 16
NUM_WORKERS = NUM_CORES * NUM_SUBCORES
BW = BATCH // NUM_WORKERS
JC = 128
NJC = BW // JC

_MESH = plsc.VectorSubcoreMesh(
    core_axis_name="c", subcore_axis_name="s", num_cores=NUM_CORES
)


@functools.partial(
    pl.kernel,
    out_type=jax.ShapeDtypeStruct((BATCH,), jnp.float32),
    mesh=_MESH,
    compiler_params=pltpu.CompilerParams(
        needs_layout_passes=False, use_tc_tiling_on_sc=False
    ),
    scratch_types=[
        pltpu.VMEM((NJC, JC), jnp.int32),
        pltpu.VMEM((NJC, JC), jnp.int32),
        pltpu.VMEM((NJC, JC), jnp.int32),
        pltpu.VMEM((DIM, BW), jnp.float32),
        pltpu.VMEM((DIM, BW), jnp.float32),
        pltpu.VMEM((DIM, BW), jnp.float32),
        pltpu.VMEM((DIM, BW), jnp.float32),
        pltpu.VMEM((DIM, BW), jnp.float32),
        pltpu.VMEM((DIM, BW), jnp.float32),
        pltpu.VMEM((BW,), jnp.float32),
        pltpu.SemaphoreType.DMA,
    ],
)
def _complex_score_sc(
    h_hbm,
    r_hbm,
    t_hbm,
    ent_re_hbm,
    ent_im_hbm,
    rel_re_hbm,
    rel_im_hbm,
    out_hbm,
    hv,
    rv,
    tv,
    hre,
    him,
    rre,
    rim,
    tre,
    tim,
    scores,
    sem,
):
    wid = lax.axis_index("s") * NUM_CORES + lax.axis_index("c")
    base = wid * BW

    for jc in range(NJC):
        off = base + jc * JC
        pltpu.sync_copy(h_hbm.at[pl.ds(off, JC)], hv.at[jc])
        pltpu.sync_copy(r_hbm.at[pl.ds(off, JC)], rv.at[jc])
        pltpu.sync_copy(t_hbm.at[pl.ds(off, JC)], tv.at[jc])

    for jc in range(NJC):
        dst = pl.ds(jc * JC, JC)

        def gbody(d, _, jc=jc, dst=dst):
            cs = [
                pltpu.async_copy(ent_re_hbm.at[d].at[hv.at[jc]], hre.at[d, dst], sem),
                pltpu.async_copy(ent_im_hbm.at[d].at[hv.at[jc]], him.at[d, dst], sem),
                pltpu.async_copy(rel_re_hbm.at[d].at[rv.at[jc]], rre.at[d, dst], sem),
                pltpu.async_copy(rel_im_hbm.at[d].at[rv.at[jc]], rim.at[d, dst], sem),
                pltpu.async_copy(ent_re_hbm.at[d].at[tv.at[jc]], tre.at[d, dst], sem),
                pltpu.async_copy(ent_im_hbm.at[d].at[tv.at[jc]], tim.at[d, dst], sem),
            ]
            for c in cs:
                c.wait()
            return 0

        lax.fori_loop(0, DIM, gbody, 0)

    def body(g, _):
        sl = pl.ds(g * 16, 16)
        acc = jnp.zeros((16,), jnp.float32)
        for d in range(DIM):
            a = hre[d, sl]
            b = him[d, sl]
            cr = rre[d, sl]
            ci = rim[d, sl]
            e = tre[d, sl]
            f = tim[d, sl]
            acc = acc + cr * (a * e + b * f) + ci * (a * f - b * e)
        scores[sl] = acc
        return 0

    lax.fori_loop(0, BW // 16, body, 0)
    pltpu.sync_copy(scores, out_hbm.at[pl.ds(base, BW)])


def kernel(triples, ent_real, ent_imag, rel_real, rel_imag):
    h = jnp.asarray(triples[:, 0], jnp.int32)
    r = jnp.asarray(triples[:, 1], jnp.int32)
    t = jnp.asarray(triples[:, 2], jnp.int32)
    return _complex_score_sc(
        h, r, t, ent_real.T, ent_imag.T, rel_real.T, rel_imag.T
    )


# speedup vs baseline: 6.0375x; 6.0375x over previous
import functools

import jax
import jax.numpy as jnp
from jax import lax
from jax.experimental import pallas as pl
from jax.experimental.pallas import tpu as pltpu
from jax.experimental.pallas import tpu_sc as plsc

BATCH = 16384
DIM = 32
NUM_CORES = 2
NUM_SUBCORES = 16
NUM_WORKERS = NUM_CORES * NUM_SUBCORES
ROWS_PER_WORKER = BATCH // NUM_WORKERS
CHUNK = 128
NUM_CHUNKS = ROWS_PER_WORKER // CHUNK
GROUPS_PER_CHUNK = CHUNK // 16

_MESH = plsc.VectorSubcoreMesh(
    core_axis_name="c", subcore_axis_name="s", num_cores=NUM_CORES
)


@functools.partial(
    pl.kernel,
    out_type=jax.ShapeDtypeStruct((BATCH,), jnp.float32),
    mesh=_MESH,
    compiler_params=pltpu.CompilerParams(
        needs_layout_passes=False, use_tc_tiling_on_sc=False
    ),
    scratch_types=[
        pltpu.VMEM((NUM_CHUNKS, CHUNK), jnp.int32),
        pltpu.VMEM((NUM_CHUNKS, CHUNK), jnp.int32),
        pltpu.VMEM((NUM_CHUNKS, CHUNK), jnp.int32),
        pltpu.VMEM((ROWS_PER_WORKER, DIM), jnp.float32),
        pltpu.VMEM((ROWS_PER_WORKER, DIM), jnp.float32),
        pltpu.VMEM((ROWS_PER_WORKER, DIM), jnp.float32),
        pltpu.VMEM((ROWS_PER_WORKER, DIM), jnp.float32),
        pltpu.VMEM((ROWS_PER_WORKER, DIM), jnp.float32),
        pltpu.VMEM((ROWS_PER_WORKER, DIM), jnp.float32),
        pltpu.VMEM((ROWS_PER_WORKER,), jnp.float32),
        pltpu.SemaphoreType.DMA,
        pltpu.SemaphoreType.DMA,
        pltpu.SemaphoreType.DMA,
        pltpu.SemaphoreType.DMA,
    ],
)
def _complex_score_sc(
    h_hbm,
    r_hbm,
    t_hbm,
    ent_re_hbm,
    ent_im_hbm,
    rel_re_hbm,
    rel_im_hbm,
    out_hbm,
    hv,
    rv,
    tv,
    hre,
    him,
    rre,
    rim,
    tre,
    tim,
    scores,
    *sems,
):
    wid = lax.axis_index("s") * NUM_CORES + lax.axis_index("c")
    base = wid * ROWS_PER_WORKER

    copies = []
    for k in range(NUM_CHUNKS):
        off = base + k * CHUNK
        pltpu.sync_copy(h_hbm.at[pl.ds(off, CHUNK)], hv.at[k])
        pltpu.sync_copy(r_hbm.at[pl.ds(off, CHUNK)], rv.at[k])
        pltpu.sync_copy(t_hbm.at[pl.ds(off, CHUNK)], tv.at[k])
        dst = pl.ds(k * CHUNK, CHUNK)
        copies.append(
            [
                pltpu.async_copy(ent_re_hbm.at[hv.at[k]], hre.at[dst], sems[k]),
                pltpu.async_copy(ent_im_hbm.at[hv.at[k]], him.at[dst], sems[k]),
                pltpu.async_copy(rel_re_hbm.at[rv.at[k]], rre.at[dst], sems[k]),
                pltpu.async_copy(rel_im_hbm.at[rv.at[k]], rim.at[dst], sems[k]),
                pltpu.async_copy(ent_re_hbm.at[tv.at[k]], tre.at[dst], sems[k]),
                pltpu.async_copy(ent_im_hbm.at[tv.at[k]], tim.at[dst], sems[k]),
            ]
        )

    lanes = lax.iota(jnp.int32, 16)

    for k in range(NUM_CHUNKS):
        for c in copies[k]:
            c.wait()

        def group_body(g, _, k=k):
            rows = k * CHUNK + g * 16 + lanes
            acc = jnp.zeros((16,), jnp.float32)
            for d in range(DIM):
                dv = jnp.full((16,), d, jnp.int32)
                a = plsc.load_gather(hre, [rows, dv])
                b = plsc.load_gather(him, [rows, dv])
                cr = plsc.load_gather(rre, [rows, dv])
                ci = plsc.load_gather(rim, [rows, dv])
                e = plsc.load_gather(tre, [rows, dv])
                f = plsc.load_gather(tim, [rows, dv])
                acc = acc + cr * (a * e + b * f) + ci * (a * f - b * e)
            scores[pl.ds(k * CHUNK + g * 16, 16)] = acc
            return 0

        lax.fori_loop(0, GROUPS_PER_CHUNK, group_body, 0)

    pltpu.sync_copy(scores, out_hbm.at[pl.ds(base, ROWS_PER_WORKER)])


def kernel(triples, ent_real, ent_imag, rel_real, rel_imag):
    h = jnp.asarray(triples[:, 0], jnp.int32)
    r = jnp.asarray(triples[:, 1], jnp.int32)
    t = jnp.asarray(triples[:, 2], jnp.int32)
    return _complex_score_sc(h, r, t, ent_real, ent_imag, rel_real, rel_imag)
